# Initial kernel scaffold; baseline (speedup 1.0000x reference)
import jax
import jax.numpy as jnp
from jax.experimental import pallas as pl


def kernel(query_length, key_length, table):
    raise NotImplementedError("write your pallas kernel here")


# speedup vs baseline: 42.7881x; 42.7881x over previous
import math

import jax
import jax.numpy as jnp
from jax import lax
from jax.experimental import pallas as pl
from jax.experimental.pallas import tpu as pltpu
from jax.experimental.pallas import tpu_sc as plsc

NUM_BUCKETS = 32
NUM_HEADS = 16
Q = 2048
K = 2048
NB = NUM_BUCKETS // 2
MAX_EXACT = NB // 2
MAX_DISTANCE = 128

TPAD = 4224
W8ROW = 4096
W8HEAD = 8 * W8ROW
BATCH = 16


def _tc_diag_body(shift_ref, table_ref, out_ref):
    shift = shift_ref[0]
    t = lax.broadcasted_iota(jnp.int32, (1, TPAD), 1)
    d = t - (Q - 1) + shift
    buckets = (d > 0).astype(jnp.int32) * NB
    rp = jnp.abs(d)
    is_small = rp < MAX_EXACT
    rp_safe = jnp.maximum(rp, 1)
    rp_if_large = MAX_EXACT + (
        jnp.log(rp_safe.astype(jnp.float32) / MAX_EXACT)
        / math.log(MAX_DISTANCE / MAX_EXACT)
        * (NB - MAX_EXACT)
    ).astype(jnp.int32)
    rp_if_large = jnp.minimum(rp_if_large, NB - 1)
    b = buckets + jnp.where(is_small, rp, rp_if_large)
    table = table_ref[...]
    w = jnp.zeros((NUM_HEADS, TPAD), jnp.float32)
    for v in range(NUM_BUCKETS):
        tcol = table[v, :].reshape(NUM_HEADS, 1)
        w = jnp.where(b == v, tcol, w)
    for r in range(8):
        out_ref[:, r, :] = w[:, r:r + W8ROW]


def _sc_body(w8_hbm, out_hbm, w8_v, sem):
    head = lax.axis_index("s")
    half = lax.axis_index("c")

    pltpu.sync_copy(
        w8_hbm.at[pl.ds(pl.multiple_of(head * W8HEAD, 8), W8HEAD)], w8_v
    )

    base_row = half * (Q // 2)

    def row_group(g, carry):
        cps = []
        for u in range(BATCH):
            i = base_row + g * BATCH + u
            srow = (Q - 1) - i
            r = srow & 7
            src = pl.multiple_of(r * W8ROW + (srow - r), 8)
            dst = pl.multiple_of((head * Q + i) * K, 8)
            cps.append(
                pltpu.async_copy(
                    w8_v.at[pl.ds(src, K)], out_hbm.at[pl.ds(dst, K)], sem
                )
            )
        for cp in cps:
            cp.wait()
        return carry

    lax.fori_loop(0, (Q // 2) // BATCH, row_group, 0)


def kernel(query_length, key_length, table):
    shift = (jnp.asarray(key_length, jnp.int32) - K) - (
        jnp.asarray(query_length, jnp.int32) - Q
    )
    shift_arr = jnp.reshape(shift, (1,))

    w8 = pl.pallas_call(
        _tc_diag_body,
        out_shape=jax.ShapeDtypeStruct((NUM_HEADS, 8, W8ROW), jnp.float32),
        in_specs=[
            pl.BlockSpec(memory_space=pltpu.SMEM),
            pl.BlockSpec(),
        ],
    )(shift_arr, table)
    w8_flat = w8.reshape(NUM_HEADS * W8HEAD)

    mesh = plsc.VectorSubcoreMesh(core_axis_name="c", subcore_axis_name="s")
    out_flat = pl.kernel(
        _sc_body,
        out_type=jax.ShapeDtypeStruct((NUM_HEADS * Q * K,), jnp.float32),
        mesh=mesh,
        scratch_types=[
            pltpu.VMEM((W8HEAD,), jnp.float32),
            pltpu.SemaphoreType.DMA,
        ],
    )(w8_flat)
    return out_flat.reshape(1, NUM_HEADS, Q, K)
